# baseline (speedup 1.0000x reference)
import functools

import jax
import jax.numpy as jnp
from jax import lax
from jax.experimental import pallas as pl
from jax.experimental.pallas import tpu as pltpu
from jax.experimental.pallas import tpu_sc as plsc

B = 4096
L_ITEM = 20
L_CTX = 50
EMBED = 32
CTXD = 32

NC = 2
NS = 16
NW = NC * NS

ITEM_PW = B * L_ITEM // NW
CTX_PW = B * L_CTX // NW
CH = 128
ITEM_CHUNKS = ITEM_PW // CH
CTX_CHUNKS = CTX_PW // CH

@functools.cache
def _sc_gather_fn():
    mesh = plsc.VectorSubcoreMesh(core_axis_name="c", subcore_axis_name="s")

    @functools.partial(
        pl.kernel,
        mesh=mesh,
        out_type=(
            jax.ShapeDtypeStruct((B * L_ITEM, EMBED), jnp.float32),
            jax.ShapeDtypeStruct((B * L_CTX, CTXD), jnp.float32),
        ),
        scratch_types=[
            pltpu.VMEM((CH,), jnp.int32),
            pltpu.VMEM((CH, EMBED), jnp.float32),
            pltpu.SemaphoreType.DMA,
        ],
        compiler_params=pltpu.CompilerParams(use_tc_tiling_on_sc=False),
    )
    def _sc_gather(item_idx, ctx_idx, item_tab, set_tab, item_out, ctx_out,
                   idx_v, rows_v, sem):
        wid = lax.axis_index("s") * NC + lax.axis_index("c")

        def item_body(j, carry):
            base = wid * ITEM_PW + j * CH
            pltpu.sync_copy(item_idx.at[pl.ds(base, CH)], idx_v)
            pltpu.async_copy(item_tab.at[idx_v], rows_v, sem).wait()
            pltpu.sync_copy(rows_v, item_out.at[pl.ds(base, CH)])
            return carry

        lax.fori_loop(0, ITEM_CHUNKS, item_body, 0)

        def ctx_body(j, carry):
            base = wid * CTX_PW + j * CH
            pltpu.sync_copy(ctx_idx.at[pl.ds(base, CH)], idx_v)
            for t in range(CH // 16):
                v = idx_v[pl.ds(t * 16, 16)]
                idx_v[pl.ds(t * 16, 16)] = jnp.maximum(v - 1, 0)
            pltpu.async_copy(set_tab.at[idx_v], rows_v, sem).wait()
            pltpu.sync_copy(rows_v, ctx_out.at[pl.ds(base, CH)])
            return carry

        lax.fori_loop(0, CTX_CHUNKS, ctx_body, 0)

    return _sc_gather


BB = 256


def _tc_body(idx_ref, it_ref, cr_ref, w1, b1, w2, b2, w3, b3, out_ref):
    maskf = (idx_ref[...] > 0).astype(jnp.float32)
    rows = cr_ref[...]
    summed = jnp.sum(rows * maskf, axis=1)
    sq = jnp.sum(summed * summed, axis=-1, keepdims=True)
    normalized = summed * lax.rsqrt(jnp.maximum(sq, 1e-4))
    h = jnp.maximum(
        jnp.dot(normalized, w1[...], preferred_element_type=jnp.float32) + b1[...], 0.0)
    h = jnp.maximum(
        jnp.dot(h, w2[...], preferred_element_type=jnp.float32) + b2[...], 0.0)
    ce = jnp.dot(h, w3[...], preferred_element_type=jnp.float32) + b3[...]
    it = it_ref[...]
    diff = it - ce[:, None, :]
    d = jnp.sqrt(jnp.sum(diff * diff, axis=-1))
    out_ref[...] = 1.0 - jnp.tanh(d)


def _tc_compute(ctx_idx3, item_rows, ctx_rows, W1, b1, W2, b2, W3, b3):
    grid = (B // BB,)
    return pl.pallas_call(
        _tc_body,
        grid=grid,
        in_specs=[
            pl.BlockSpec((BB, L_CTX, 1), lambda i: (i, 0, 0)),
            pl.BlockSpec((BB, L_ITEM, EMBED), lambda i: (i, 0, 0)),
            pl.BlockSpec((BB, L_CTX, CTXD), lambda i: (i, 0, 0)),
            pl.BlockSpec((CTXD, 2 * CTXD), lambda i: (0, 0)),
            pl.BlockSpec((1, 2 * CTXD), lambda i: (0, 0)),
            pl.BlockSpec((2 * CTXD, 4 * CTXD), lambda i: (0, 0)),
            pl.BlockSpec((1, 4 * CTXD), lambda i: (0, 0)),
            pl.BlockSpec((4 * CTXD, EMBED), lambda i: (0, 0)),
            pl.BlockSpec((1, EMBED), lambda i: (0, 0)),
        ],
        out_specs=pl.BlockSpec((BB, L_ITEM), lambda i: (i, 0)),
        out_shape=jax.ShapeDtypeStruct((B, L_ITEM), jnp.float32),
    )(ctx_idx3, item_rows, ctx_rows, W1, b1, W2, b2, W3, b3)


def kernel(item_indices, context_indices, item_table, set_table,
           W1, b1, W2, b2, W3, b3):
    item_rows, ctx_rows = _sc_gather_fn()(
        item_indices.reshape(-1), context_indices.reshape(-1),
        item_table, set_table)
    return _tc_compute(
        context_indices.reshape(B, L_CTX, 1),
        item_rows.reshape(B, L_ITEM, EMBED),
        ctx_rows.reshape(B, L_CTX, CTXD),
        W1, b1.reshape(1, -1), W2, b2.reshape(1, -1), W3, b3.reshape(1, -1))
